# baseline (speedup 1.0000x reference)
import jax
import jax.numpy as jnp
from jax import lax
from jax.experimental import pallas as pl
from jax.experimental.pallas import tpu as pltpu
from jax.experimental.pallas import tpu_sc as plsc

N_FIELDS = 26
VOCAB1 = 100001
EMBED_DIM = 32
BATCH = 16384

NUM_CORES = 2
NUM_SUBCORES = 16
NW = NUM_CORES * NUM_SUBCORES
TOTAL_ROWS = BATCH * N_FIELDS
ROWS_PER_W = TOTAL_ROWS // NW
CHUNK = 1664
NCHUNK = ROWS_PER_W // CHUNK
DMAS = CHUNK // 128
VREGS = CHUNK // 16


def _body(x_hbm, tab_hbm, out_hbm, idx_v, gidx_v, rows_v, off_v, zbuf, sem):
    wid = lax.axis_index("s") * NUM_CORES + lax.axis_index("c")
    lanes = lax.iota(jnp.int32, 16)

    for v in range(VREGS):
        j, k = divmod(v, 8)
        lo = v * 16
        f = lanes + (lo - (lo // 26) * 26)
        f = jnp.where(f >= N_FIELDS, f - N_FIELDS, f)
        off_v[j, pl.ds(k * 16, 16)] = f * VOCAB1

    def chunk_body(c, carry):
        pltpu.sync_copy(x_hbm.at[wid, c], idx_v)

        def gidx_body(j, zv):
            for k in range(8):
                sl = pl.ds(k * 16, 16)
                iv = idx_v[j, sl]
                gidx_v[j, sl] = iv + off_v[j, sl]
                zv = jnp.minimum(zv, iv)
            return zv

        zv = lax.fori_loop(0, DMAS, gidx_body,
                           jnp.full((16,), 1, jnp.int32))

        descs = [
            pltpu.async_copy(tab_hbm.at[gidx_v.at[j]],
                             rows_v.at[pl.ds(j * 128, 128)], sem)
            for j in range(DMAS)
        ]
        for d in descs:
            d.wait()

        s = zv[0]
        for i in range(1, 16):
            s = jnp.minimum(s, zv[i])

        @pl.when(s == 0)
        def _fixup():
            z = jnp.zeros((16,), jnp.float32)

            def fix_body(j, _):
                for k in range(8):
                    iv = idx_v[j, pl.ds(k * 16, 16)]
                    t = iv[0]
                    for i in range(1, 16):
                        t = jnp.minimum(t, iv[i])

                    @pl.when(t == 0)
                    def _dirty_vreg():
                        for i in range(16):
                            @pl.when(iv[i] == 0)
                            def _zero_row(i=i):
                                row = j * 128 + k * 16 + i
                                rows_v[row, pl.ds(0, 16)] = z
                                rows_v[row, pl.ds(16, 16)] = z
                return 0

            lax.fori_loop(0, DMAS, fix_body, 0)

        pltpu.sync_copy(rows_v, out_hbm.at[wid, c])
        return carry

    lax.fori_loop(0, NCHUNK, chunk_body, 0)


@jax.jit
def kernel(x, tables):
    x_r = x.astype(jnp.int32).reshape(NW, NCHUNK, DMAS, 128)
    tab = tables.reshape(N_FIELDS * VOCAB1, EMBED_DIM)
    emb = pl.kernel(
        _body,
        out_type=jax.ShapeDtypeStruct((NW, NCHUNK, CHUNK, EMBED_DIM),
                                      jnp.float32),
        mesh=plsc.VectorSubcoreMesh(core_axis_name="c", subcore_axis_name="s",
                                    num_cores=NUM_CORES,
                                    num_subcores=NUM_SUBCORES),
        scratch_types=[
            pltpu.VMEM((DMAS, 128), jnp.int32),
            pltpu.VMEM((DMAS, 128), jnp.int32),
            pltpu.VMEM((CHUNK, EMBED_DIM), jnp.float32),
            pltpu.VMEM((DMAS, 128), jnp.int32),
            pltpu.VMEM((16,), jnp.int32),
            pltpu.SemaphoreType.DMA,
        ],
        compiler_params=pltpu.CompilerParams(use_tc_tiling_on_sc=False),
    )
    out = emb(x_r, tab)
    return out.reshape(BATCH, N_FIELDS * EMBED_DIM)


# speedup vs baseline: 1.0010x; 1.0010x over previous
import jax
import jax.numpy as jnp
from jax import lax
from jax.experimental import pallas as pl
from jax.experimental.pallas import tpu as pltpu
from jax.experimental.pallas import tpu_sc as plsc

N_FIELDS = 26
VOCAB1 = 100001
EMBED_DIM = 32
BATCH = 16384

NUM_CORES = 2
NUM_SUBCORES = 16
NW = NUM_CORES * NUM_SUBCORES
TOTAL_ROWS = BATCH * N_FIELDS
ROWS_PER_W = TOTAL_ROWS // NW
CHUNK = 1664
NCHUNK = ROWS_PER_W // CHUNK
DMAS = CHUNK // 128
VREGS = CHUNK // 16


def _body(x_hbm, tab_hbm, out_hbm, idx_v, gidx_v, rows_v, off_v, zbuf, sem):
    wid = lax.axis_index("s") * NUM_CORES + lax.axis_index("c")
    lanes = lax.iota(jnp.int32, 16)

    for v in range(VREGS):
        j, k = divmod(v, 8)
        lo = v * 16
        f = lanes + (lo - (lo // 26) * 26)
        f = jnp.where(f >= N_FIELDS, f - N_FIELDS, f)
        off_v[j, pl.ds(k * 16, 16)] = f * VOCAB1

    def chunk_body(c, carry):
        pltpu.sync_copy(x_hbm.at[pl.ds((wid * NCHUNK + c) * DMAS, DMAS)],
                        idx_v)

        def gidx_body(j, zv):
            for k in range(8):
                sl = pl.ds(k * 16, 16)
                iv = idx_v[j, sl]
                gidx_v[j, sl] = iv + off_v[j, sl]
                zv = jnp.minimum(zv, iv)
            return zv

        zv = lax.fori_loop(0, DMAS, gidx_body,
                           jnp.full((16,), 1, jnp.int32))

        descs = [
            pltpu.async_copy(tab_hbm.at[gidx_v.at[j]],
                             rows_v.at[pl.ds(j * 128, 128)], sem)
            for j in range(DMAS)
        ]
        for d in descs:
            d.wait()

        s = zv[0]
        for i in range(1, 16):
            s = jnp.minimum(s, zv[i])

        @pl.when(s == 0)
        def _fixup():
            z = jnp.zeros((16,), jnp.float32)

            def fix_body(j, _):
                for k in range(8):
                    iv = idx_v[j, pl.ds(k * 16, 16)]
                    t = iv[0]
                    for i in range(1, 16):
                        t = jnp.minimum(t, iv[i])

                    @pl.when(t == 0)
                    def _dirty_vreg():
                        for i in range(16):
                            @pl.when(iv[i] == 0)
                            def _zero_row(i=i):
                                row = j * 128 + k * 16 + i
                                rows_v[row, pl.ds(0, 16)] = z
                                rows_v[row, pl.ds(16, 16)] = z
                return 0

            lax.fori_loop(0, DMAS, fix_body, 0)

        pltpu.sync_copy(rows_v,
                        out_hbm.at[pl.ds((wid * NCHUNK + c) * CHUNK, CHUNK)])
        return carry

    lax.fori_loop(0, NCHUNK, chunk_body, 0)


@jax.jit
def kernel(x, tables):
    x_r = x.astype(jnp.int32).reshape(TOTAL_ROWS // 128, 128)
    tab = tables.reshape(N_FIELDS * VOCAB1, EMBED_DIM)
    emb = pl.kernel(
        _body,
        out_type=jax.ShapeDtypeStruct((TOTAL_ROWS, EMBED_DIM), jnp.float32),
        mesh=plsc.VectorSubcoreMesh(core_axis_name="c", subcore_axis_name="s",
                                    num_cores=NUM_CORES,
                                    num_subcores=NUM_SUBCORES),
        scratch_types=[
            pltpu.VMEM((DMAS, 128), jnp.int32),
            pltpu.VMEM((DMAS, 128), jnp.int32),
            pltpu.VMEM((CHUNK, EMBED_DIM), jnp.float32),
            pltpu.VMEM((DMAS, 128), jnp.int32),
            pltpu.VMEM((16,), jnp.int32),
            pltpu.SemaphoreType.DMA,
        ],
        compiler_params=pltpu.CompilerParams(use_tc_tiling_on_sc=False),
    )
    out = emb(x_r, tab)
    return out.reshape(BATCH, N_FIELDS * EMBED_DIM)


# speedup vs baseline: 30.0418x; 30.0128x over previous
import jax
import jax.numpy as jnp
from jax import lax
from jax.experimental import pallas as pl
from jax.experimental.pallas import tpu as pltpu
from jax.experimental.pallas import tpu_sc as plsc

N_FIELDS = 26
VOCAB1 = 100001
EMBED_DIM = 32
BATCH = 16384

NUM_CORES = 2
NUM_SUBCORES = 16
NW = NUM_CORES * NUM_SUBCORES
ROWS = N_FIELDS * EMBED_DIM
ROWS_PER_W = ROWS // NW
OUT_CHUNK = 4096
N_OCHUNK = BATCH // OUT_CHUNK
VPC = OUT_CHUNK // 16


def _body(xt_hbm, tab_hbm, out_hbm, idx_v, row_v, out_v):
    wid = lax.axis_index("s") * NUM_CORES + lax.axis_index("c")

    def row_body(r, carry):
        fd = wid * ROWS_PER_W + r
        f = fd // EMBED_DIM
        pltpu.sync_copy(tab_hbm.at[fd], row_v)
        pltpu.sync_copy(xt_hbm.at[f], idx_v)

        for c in range(N_OCHUNK):
            def gather_body(v, _):
                iv = idx_v[pl.ds(c * OUT_CHUNK + v * 16, 16)]
                g = plsc.load_gather(row_v, [iv])
                out_v[pl.ds(v * 16, 16)] = jnp.where(iv == 0, 0.0, g)
                return 0

            lax.fori_loop(0, VPC, gather_body, 0)
            pltpu.sync_copy(out_v, out_hbm.at[fd, pl.ds(c * OUT_CHUNK,
                                                        OUT_CHUNK)])
        return carry

    lax.fori_loop(0, ROWS_PER_W, row_body, 0)


@jax.jit
def kernel(x, tables):
    xt = x.astype(jnp.int32).T
    tab = tables.transpose(0, 2, 1).reshape(ROWS, VOCAB1)
    emb = pl.kernel(
        _body,
        out_type=jax.ShapeDtypeStruct((ROWS, BATCH), jnp.float32),
        mesh=plsc.VectorSubcoreMesh(core_axis_name="c", subcore_axis_name="s",
                                    num_cores=NUM_CORES,
                                    num_subcores=NUM_SUBCORES),
        scratch_types=[
            pltpu.VMEM((BATCH,), jnp.int32),
            pltpu.VMEM((VOCAB1,), jnp.float32),
            pltpu.VMEM((OUT_CHUNK,), jnp.float32),
        ],
        compiler_params=pltpu.CompilerParams(use_tc_tiling_on_sc=True,
                                             needs_layout_passes=False),
    )
    out_t = emb(xt, tab)
    return out_t.T


# speedup vs baseline: 31.5344x; 1.0497x over previous
import jax
import jax.numpy as jnp
from jax import lax
from jax.experimental import pallas as pl
from jax.experimental.pallas import tpu as pltpu
from jax.experimental.pallas import tpu_sc as plsc

N_FIELDS = 26
VOCAB1 = 100001
EMBED_DIM = 32
BATCH = 16384

NUM_CORES = 2
NUM_SUBCORES = 16
NW = NUM_CORES * NUM_SUBCORES
ROWS = N_FIELDS * EMBED_DIM
ROWS_PER_W = ROWS // NW
OUT_CHUNK = 4096
N_OCHUNK = BATCH // OUT_CHUNK
VPC = OUT_CHUNK // 16


def _body(xt_hbm, tab_hbm, out_hbm, idx_v, row_v, out_v, sem0, sem1):
    wid = lax.axis_index("s") * NUM_CORES + lax.axis_index("c")
    lanes = lax.iota(jnp.int32, 16)
    sems = (sem0, sem1)
    descs = [None, None]
    f_prev = None

    for r in range(ROWS_PER_W):
        fd = wid * ROWS_PER_W + r
        f = fd // EMBED_DIM
        pltpu.sync_copy(tab_hbm.at[fd], row_v)
        if f_prev is None:
            pltpu.sync_copy(xt_hbm.at[f], idx_v)
        else:
            @pl.when(f != f_prev)
            def _load_idx(f=f):
                pltpu.sync_copy(xt_hbm.at[f], idx_v)
        f_prev = f

        v0 = row_v[pl.ds(0, 16)]
        row_v[pl.ds(0, 16)] = jnp.where(lanes == 0, 0.0, v0)

        for c in range(N_OCHUNK):
            b = (r * N_OCHUNK + c) % 2
            if descs[b] is not None:
                descs[b].wait()

            def gather_body(v, _, c=c, b=b):
                iv = idx_v[pl.ds(c * OUT_CHUNK + v * 16, 16)]
                out_v[b, pl.ds(v * 16, 16)] = plsc.load_gather(row_v, [iv])
                return 0

            lax.fori_loop(0, VPC, gather_body, 0)
            descs[b] = pltpu.async_copy(
                out_v.at[b], out_hbm.at[fd, pl.ds(c * OUT_CHUNK, OUT_CHUNK)],
                sems[b])

    for d in descs:
        if d is not None:
            d.wait()


@jax.jit
def kernel(x, tables):
    xt = x.astype(jnp.int32).T
    tab = tables.transpose(0, 2, 1).reshape(ROWS, VOCAB1)
    emb = pl.kernel(
        _body,
        out_type=jax.ShapeDtypeStruct((ROWS, BATCH), jnp.float32),
        mesh=plsc.VectorSubcoreMesh(core_axis_name="c", subcore_axis_name="s",
                                    num_cores=NUM_CORES,
                                    num_subcores=NUM_SUBCORES),
        scratch_types=[
            pltpu.VMEM((BATCH,), jnp.int32),
            pltpu.VMEM((VOCAB1,), jnp.float32),
            pltpu.VMEM((2, OUT_CHUNK), jnp.float32),
            pltpu.SemaphoreType.DMA,
            pltpu.SemaphoreType.DMA,
        ],
        compiler_params=pltpu.CompilerParams(use_tc_tiling_on_sc=True,
                                             needs_layout_passes=False),
    )
    out_t = emb(xt, tab)
    return out_t.T
